# baseline (speedup 1.0000x reference)
import functools

import jax
import jax.numpy as jnp
import numpy as np
from jax.experimental import pallas as pl
from jax.experimental.pallas import tpu as pltpu

_NPTS = 64
_LANES = 128
_RQ = 56


def _kernel_body(params_ref, depth_ref, x_ref, y_ref, xyp_ref, w_ref, cm_ref,
                 zs_ref, dirs_ref, org_ref, xy_ref):
    rq = depth_ref.shape[1]

    fx = params_ref[0, 0, 0]
    fy = params_ref[0, 0, 1]
    px = params_ref[0, 0, 2]
    py = params_ref[0, 0, 3]
    r = [[params_ref[0, 0, 4 + 3 * i + j] for j in range(3)]
         for i in range(3)]
    t = [params_ref[0, 0, 13 + j] for j in range(3)]

    d_blk = depth_ref[0]
    eye = (jax.lax.broadcasted_iota(jnp.int32, (_LANES, _LANES), 0) ==
           jax.lax.broadcasted_iota(jnp.int32, (_LANES, _LANES), 1)
           ).astype(jnp.float32)
    d_t = jax.lax.dot_general(eye, d_blk, (((1,), (1,)), ((), ())),
                              preferred_element_type=jnp.float32)
    kline = (jax.lax.broadcasted_iota(jnp.int32, (_LANES, _NPTS), 1)
             .astype(jnp.float32)
             * jnp.float32(1.0 / (_NPTS - 1)) + jnp.float32(0.5))
    for i in range(rq):
        zs_ref[0, i] = d_t[:, i:i + 1] * kline

    ux = (x_ref[...] - px) / fx
    uy = (y_ref[...] - py) / fy
    dx = ux * r[0][0] + uy * r[0][1] + r[0][2]
    dy = ux * r[1][0] + uy * r[1][1] + r[1][2]
    dz = ux * r[2][0] + uy * r[2][1] + r[2][2]
    inv = jax.lax.rsqrt(dx * dx + dy * dy + dz * dz)
    s_cat = jnp.concatenate([dx * inv, dy * inv, dz * inv], axis=1)
    for s in range(3):
        dirs_ref[0, :, s, :] = jnp.dot(s_cat, w_ref[s],
                                       preferred_element_type=jnp.float32)

    o = [-(t[0] * r[i][0] + t[1] * r[i][1] + t[2] * r[i][2]) for i in range(3)]
    for s in range(3):
        cms = cm_ref[s:s + 1, :]
        row = jnp.where(cms == 0.0, o[0], jnp.where(cms == 1.0, o[1], o[2]))
        org_ref[0, :, s, :] = jnp.broadcast_to(row, (rq, _LANES))

    xy_ref[0] = xyp_ref[...]


@functools.cache
def _spread_select_w(n_chunk_vals: int):
    w = np.zeros((3, 3 * _LANES, _LANES), np.float32)
    for s in range(3):
        for l in range(_LANES):
            m = _LANES * s + l
            c, p = m % 3, m // 3
            w[s, c * _LANES + p, l] = 1.0
    return jnp.asarray(w)


@functools.cache
def _cyc3_pattern():
    cm = np.zeros((3, _LANES), np.float32)
    for s in range(3):
        for l in range(_LANES):
            cm[s, l] = (_LANES * s + l) % 3
    return jnp.asarray(cm)


@jax.jit
def _run(depth_channel, R, T, focal, principal, xy_grid):
    B_, H_, W_ = depth_channel.shape
    n = H_ * W_
    nq = n // _LANES
    rq = _RQ
    grid = (nq // rq, B_)

    depth_in = depth_channel.reshape(B_, nq, _LANES)
    x_pl = xy_grid[:, :, 0].reshape(nq, _LANES)
    y_pl = xy_grid[:, :, 1].reshape(nq, _LANES)
    xy_pairs = xy_grid.reshape(nq, 2 * _LANES)
    params = jnp.concatenate(
        [focal, principal, R.reshape(B_, 9), T], axis=1).reshape(B_, 1, 16)
    w_mat = _spread_select_w(_LANES)
    cm = _cyc3_pattern()

    zs, dirs, org, xy = pl.pallas_call(
        _kernel_body,
        grid=grid,
        in_specs=[
            pl.BlockSpec((1, 1, 16), lambda q, b: (b, 0, 0),
                         memory_space=pltpu.SMEM),
            pl.BlockSpec((1, rq, _LANES), lambda q, b: (b, q, 0)),
            pl.BlockSpec((rq, _LANES), lambda q, b: (q, 0)),
            pl.BlockSpec((rq, _LANES), lambda q, b: (q, 0)),
            pl.BlockSpec((rq, 2 * _LANES), lambda q, b: (q, 0)),
            pl.BlockSpec((3, 3 * _LANES, _LANES), lambda q, b: (0, 0, 0)),
            pl.BlockSpec((3, _LANES), lambda q, b: (0, 0)),
        ],
        out_specs=[
            pl.BlockSpec((1, rq, _LANES, _NPTS), lambda q, b: (b, q, 0, 0)),
            pl.BlockSpec((1, rq, 3, _LANES), lambda q, b: (b, q, 0, 0)),
            pl.BlockSpec((1, rq, 3, _LANES), lambda q, b: (b, q, 0, 0)),
            pl.BlockSpec((1, rq, 2 * _LANES), lambda q, b: (b, q, 0)),
        ],
        out_shape=[
            jax.ShapeDtypeStruct((B_, nq, _LANES, _NPTS), jnp.float32),
            jax.ShapeDtypeStruct((B_, nq, 3, _LANES), jnp.float32),
            jax.ShapeDtypeStruct((B_, nq, 3, _LANES), jnp.float32),
            jax.ShapeDtypeStruct((B_, nq, 2 * _LANES), jnp.float32),
        ],
        compiler_params=pltpu.CompilerParams(
            dimension_semantics=("arbitrary", "arbitrary")),
    )(params, depth_in, x_pl, y_pl, xy_pairs, w_mat, cm)

    return (org.reshape(B_, H_, W_, 3),
            dirs.reshape(B_, H_, W_, 3),
            zs.reshape(B_, H_, W_, _NPTS),
            xy.reshape(B_, H_, W_, 2))


def kernel(depth_channel, R, T, focal, principal, xy_grid):
    return _run(depth_channel, R, T, focal, principal, xy_grid)
